# Initial kernel scaffold; baseline (speedup 1.0000x reference)
import jax
import jax.numpy as jnp
from jax.experimental import pallas as pl


def kernel(anchor, gt):
    raise NotImplementedError("write your pallas kernel here")


# speedup vs baseline: 1.9462x; 1.9462x over previous
import functools

import jax
import jax.numpy as jnp
from jax.experimental import pallas as pl

POS = 0.5
NEG = 0.3
EPS = 1e-6
BIG = 2**30

N = 20000
M = 256
B = 1024
NB = 20
NPAD = NB * B


def _stage1_body(a_ref, g_ref, assign_ref, cmax_ref, carg_ref):
    i = pl.program_id(0)
    ax1 = a_ref[0:1, :]
    ay1 = a_ref[1:2, :]
    ax2 = a_ref[2:3, :]
    ay2 = a_ref[3:4, :]
    gx1 = g_ref[:, 0:1]
    gy1 = g_ref[:, 1:2]
    gx2 = g_ref[:, 2:3]
    gy2 = g_ref[:, 3:4]

    lt_x = jnp.maximum(ax1, gx1)
    lt_y = jnp.maximum(ay1, gy1)
    rb_x = jnp.minimum(ax2, gx2)
    rb_y = jnp.minimum(ay2, gy2)
    w = jnp.maximum(rb_x - lt_x, 0.0)
    h = jnp.maximum(rb_y - lt_y, 0.0)
    overlap = w * h
    area_a = (ax2 - ax1) * (ay2 - ay1)
    area_b = (gx2 - gx1) * (gy2 - gy1)
    union = area_a + area_b - overlap
    iou = overlap / jnp.maximum(union, EPS)

    maxr = jnp.max(iou, axis=0, keepdims=True)
    gids = jax.lax.broadcasted_iota(jnp.int32, (M, B), 0)
    argr = jnp.min(jnp.where(iou == maxr, gids, BIG), axis=0, keepdims=True)
    base = jnp.where(maxr < NEG, jnp.int32(-1),
                     jnp.where(maxr > POS, argr, jnp.int32(-2)))
    assign_ref[...] = base.reshape(1, 1, B)

    cmax = jnp.max(iou, axis=1, keepdims=True)
    aids = jax.lax.broadcasted_iota(jnp.int32, (M, B), 1)
    cargl = jnp.min(jnp.where(iou == cmax, aids, BIG), axis=1, keepdims=True)
    carg = cargl + i * B

    @pl.when(i == 0)
    def _init():
        cmax_ref[...] = cmax
        carg_ref[...] = carg

    @pl.when(i > 0)
    def _merge():
        upd = cmax > cmax_ref[...]
        carg_ref[...] = jnp.where(upd, carg, carg_ref[...])
        cmax_ref[...] = jnp.where(upd, cmax, cmax_ref[...])


def _stage2_body(carg_ref, base_ref, out_ref):
    i = pl.program_id(0)
    carg = carg_ref[...]
    aids = jax.lax.broadcasted_iota(jnp.int32, (M, B), 1) + i * B
    gids = jax.lax.broadcasted_iota(jnp.int32, (M, B), 0)
    wmat = jnp.where(carg == aids, gids, jnp.int32(-1))
    winner = jnp.max(wmat, axis=0, keepdims=True)
    base = base_ref[0, 0, :].reshape(1, B)
    out_ref[...] = jnp.where(winner >= 0, winner, base).reshape(1, 1, B)


def _stage1(a_t, gt):
    return pl.pallas_call(
        _stage1_body,
        grid=(NB,),
        in_specs=[
            pl.BlockSpec((4, B), lambda i: (0, i)),
            pl.BlockSpec((M, 4), lambda i: (0, 0)),
        ],
        out_specs=[
            pl.BlockSpec((1, 1, B), lambda i: (i, 0, 0)),
            pl.BlockSpec((M, 1), lambda i: (0, 0)),
            pl.BlockSpec((M, 1), lambda i: (0, 0)),
        ],
        out_shape=[
            jax.ShapeDtypeStruct((NB, 1, B), jnp.int32),
            jax.ShapeDtypeStruct((M, 1), jnp.float32),
            jax.ShapeDtypeStruct((M, 1), jnp.int32),
        ],
    )(a_t, gt)


def _stage2(carg, base):
    return pl.pallas_call(
        _stage2_body,
        grid=(NB,),
        in_specs=[
            pl.BlockSpec((M, 1), lambda i: (0, 0)),
            pl.BlockSpec((1, 1, B), lambda i: (i, 0, 0)),
        ],
        out_specs=pl.BlockSpec((1, 1, B), lambda i: (i, 0, 0)),
        out_shape=jax.ShapeDtypeStruct((NB, 1, B), jnp.int32),
    )(carg, base)


@jax.jit
def kernel(anchor, gt):
    a_t = jnp.concatenate(
        [anchor.T, jnp.full((4, NPAD - N), -1.0, dtype=anchor.dtype)], axis=1)
    base, _cmax, carg = _stage1(a_t, gt)
    out = _stage2(carg, base)
    return out.reshape(NPAD)[:N]
